# Initial kernel scaffold; baseline (speedup 1.0000x reference)
import jax
import jax.numpy as jnp
from jax.experimental import pallas as pl


def kernel(center, context, negatives, W_in, W_out):
    raise NotImplementedError("write your pallas kernel here")


# speedup vs baseline: 4.0012x; 4.0012x over previous
import functools

import jax
import jax.numpy as jnp
from jax import lax
from jax.experimental import pallas as pl
from jax.experimental.pallas import tpu as pltpu
from jax.experimental.pallas import tpu_sc as plsc

V = 1000000
D = 64
B = 16384
K = 20

NC = 2
NS = 16
L = 16
NW = NC * NS
BPW = B // NW
C = 32
NCHUNK = BPW // C
G = C // L
NEG_STREAMS = (C * K) // 128


def _sc_scores(center, context, neg_flat, W_in, W_out):
    mesh = plsc.VectorSubcoreMesh(
        core_axis_name="c", subcore_axis_name="s", num_cores=NC,
        num_subcores=NS)

    @functools.partial(
        pl.kernel,
        out_type=(
            jax.ShapeDtypeStruct((B,), jnp.float32),
            jax.ShapeDtypeStruct((B * K,), jnp.float32),
        ),
        mesh=mesh,
        scratch_types=[
            pltpu.VMEM((C,), jnp.int32),
            pltpu.VMEM((C,), jnp.int32),
            pltpu.VMEM((C * K,), jnp.int32),
            pltpu.VMEM((C, D), jnp.float32),
            pltpu.VMEM((C, D), jnp.float32),
            pltpu.VMEM((C * K, D), jnp.float32),
            pltpu.VMEM((C,), jnp.float32),
            pltpu.VMEM((C * K,), jnp.float32),
            pltpu.SemaphoreType.DMA,
        ],
        compiler_params=pltpu.CompilerParams(
            needs_layout_passes=False, use_tc_tiling_on_sc=False),
    )
    def body(cen_hbm, ctx_hbm, neg_hbm, win_hbm, wout_hbm,
             pos_out, negs_out,
             cen_idx, ctx_idx, neg_idx, cen_rows, ctx_rows, neg_rows,
             pos_buf, neg_buf, sem):
        wid = lax.axis_index("s") * NC + lax.axis_index("c")

        def chunk_body(ci, _):
            base = pl.multiple_of(wid * BPW + ci * C, C)
            nbase = pl.multiple_of(base * K, C * K)
            pltpu.sync_copy(cen_hbm.at[pl.ds(base, C)], cen_idx)
            pltpu.sync_copy(ctx_hbm.at[pl.ds(base, C)], ctx_idx)
            pltpu.sync_copy(neg_hbm.at[pl.ds(nbase, C * K)], neg_idx)
            copies = [
                pltpu.async_copy(win_hbm.at[cen_idx], cen_rows, sem),
                pltpu.async_copy(wout_hbm.at[ctx_idx], ctx_rows, sem),
            ]
            for j in range(NEG_STREAMS):
                copies.append(pltpu.async_copy(
                    wout_hbm.at[neg_idx.at[pl.ds(j * 128, 128)]],
                    neg_rows.at[pl.ds(j * 128, 128)], sem))
            for cp in copies:
                cp.wait()

            for g in range(G):
                row16 = g * L + lax.iota(jnp.int32, L)
                nrow = [row16 * K + k for k in range(K)]
                zero = jnp.zeros((L,), jnp.float32)

                def dot_step(d, carry):
                    pos = carry[0]
                    accs = list(carry[1:])
                    col = jnp.full((L,), d, jnp.int32)
                    c_d = plsc.load_gather(cen_rows, [row16, col])
                    x_d = plsc.load_gather(ctx_rows, [row16, col])
                    pos = pos + c_d * x_d
                    new = [accs[k] + c_d * plsc.load_gather(
                        neg_rows, [nrow[k], col]) for k in range(K)]
                    return (pos, *new)

                res = lax.fori_loop(0, D, dot_step,
                                    (zero,) * (K + 1), unroll=2)
                pos_buf[pl.ds(g * L, L)] = res[0]
                for k in range(K):
                    plsc.store_scatter(neg_buf, [nrow[k]], res[1 + k])

            pltpu.sync_copy(pos_buf, pos_out.at[pl.ds(base, C)])
            pltpu.sync_copy(neg_buf, negs_out.at[pl.ds(nbase, C * K)])
            return ()

        lax.fori_loop(0, NCHUNK, chunk_body, ())

    return body(center, context, neg_flat, W_in, W_out)


def _tc_loss_body(pos_ref, neg_ref, out_ref):
    p = pos_ref[...]
    n = neg_ref[...]
    s1 = jnp.sum(-jnp.log(jax.nn.sigmoid(p) + 1e-10))
    s2 = jnp.sum(-jnp.log(jax.nn.sigmoid(-n) + 1e-10))
    out_ref[...] = jnp.broadcast_to((s1 + s2) * (1.0 / B), (1, 1))


_tc_loss = pl.pallas_call(
    _tc_loss_body,
    out_shape=jax.ShapeDtypeStruct((1, 1), jnp.float32),
)


def kernel(center, context, negatives, W_in, W_out):
    center = center.astype(jnp.int32)
    context = context.astype(jnp.int32)
    neg_flat = negatives.astype(jnp.int32).reshape(-1)
    pos, negs = _sc_scores(center, context, neg_flat, W_in, W_out)
    loss = _tc_loss(pos.reshape(128, 128), negs.reshape(B * K // 128, 128))
    return loss[0, 0]


# speedup vs baseline: 5.1463x; 1.2862x over previous
import functools

import jax
import jax.numpy as jnp
from jax import lax
from jax.experimental import pallas as pl
from jax.experimental.pallas import tpu as pltpu
from jax.experimental.pallas import tpu_sc as plsc

V = 1000000
D = 64
B = 16384
K = 20

NC = 2
NS = 16
L = 16
NW = NC * NS
BPW = B // NW
C = 32
NCHUNK = BPW // C
G = C // L
NEG_STREAMS = (C * K) // 128


def _sc_scores(center, context, neg_flat, W_in, W_out):
    mesh = plsc.VectorSubcoreMesh(
        core_axis_name="c", subcore_axis_name="s", num_cores=NC,
        num_subcores=NS)

    @functools.partial(
        pl.kernel,
        out_type=(
            jax.ShapeDtypeStruct((B,), jnp.float32),
            jax.ShapeDtypeStruct((B * K,), jnp.float32),
        ),
        mesh=mesh,
        scratch_types=[
            pltpu.VMEM((C,), jnp.int32),
            pltpu.VMEM((C,), jnp.int32),
            pltpu.VMEM((C * K,), jnp.int32),
            pltpu.VMEM((C, D), jnp.float32),
            pltpu.VMEM((C, D), jnp.float32),
            pltpu.VMEM((C * K, D), jnp.float32),
            pltpu.VMEM((C,), jnp.float32),
            pltpu.VMEM((C * K,), jnp.float32),
            pltpu.SemaphoreType.DMA,
        ],
        compiler_params=pltpu.CompilerParams(
            needs_layout_passes=False, use_tc_tiling_on_sc=False),
    )
    def body(cen_hbm, ctx_hbm, neg_hbm, win_hbm, wout_hbm,
             pos_out, negs_out,
             cen_idx, ctx_idx, neg_idx, cen_rows, ctx_rows, neg_rows,
             pos_buf, neg_buf, sem):
        wid = lax.axis_index("s") * NC + lax.axis_index("c")

        def chunk_body(ci, _):
            base = pl.multiple_of(wid * BPW + ci * C, C)
            nbase = pl.multiple_of(base * K, C * K)
            pltpu.sync_copy(cen_hbm.at[pl.ds(base, C)], cen_idx)
            pltpu.sync_copy(ctx_hbm.at[pl.ds(base, C)], ctx_idx)
            pltpu.sync_copy(neg_hbm.at[pl.ds(nbase, C * K)], neg_idx)
            copies = [
                pltpu.async_copy(win_hbm.at[cen_idx], cen_rows, sem),
                pltpu.async_copy(wout_hbm.at[ctx_idx], ctx_rows, sem),
            ]
            for j in range(NEG_STREAMS):
                copies.append(pltpu.async_copy(
                    wout_hbm.at[neg_idx.at[pl.ds(j * 128, 128)]],
                    neg_rows.at[pl.ds(j * 128, 128)], sem))
            for cp in copies:
                cp.wait()

            for g in range(G):
                lane = lax.iota(jnp.int32, L)
                row16 = g * L + lane
                nrow = [row16 * K + k for k in range(K)]
                zero = jnp.zeros((L,), jnp.float32)

                def dot_step(d, carry):
                    pos = carry[0]
                    accs = list(carry[1:])
                    col = (d + lane) & (D - 1)
                    c_d = plsc.load_gather(cen_rows, [row16, col])
                    x_d = plsc.load_gather(ctx_rows, [row16, col])
                    pos = pos + c_d * x_d
                    new = [accs[k] + c_d * plsc.load_gather(
                        neg_rows, [nrow[k], col]) for k in range(K)]
                    return (pos, *new)

                res = lax.fori_loop(0, D, dot_step,
                                    (zero,) * (K + 1), unroll=2)
                pos_buf[pl.ds(g * L, L)] = res[0]
                for k in range(K):
                    plsc.store_scatter(neg_buf, [nrow[k]], res[1 + k])

            pltpu.sync_copy(pos_buf, pos_out.at[pl.ds(base, C)])
            pltpu.sync_copy(neg_buf, negs_out.at[pl.ds(nbase, C * K)])
            return ()

        lax.fori_loop(0, NCHUNK, chunk_body, ())

    return body(center, context, neg_flat, W_in, W_out)


def _tc_loss_body(pos_ref, neg_ref, out_ref):
    p = pos_ref[...]
    n = neg_ref[...]
    s1 = jnp.sum(-jnp.log(jax.nn.sigmoid(p) + 1e-10))
    s2 = jnp.sum(-jnp.log(jax.nn.sigmoid(-n) + 1e-10))
    out_ref[...] = jnp.broadcast_to((s1 + s2) * (1.0 / B), (1, 1))


_tc_loss = pl.pallas_call(
    _tc_loss_body,
    out_shape=jax.ShapeDtypeStruct((1, 1), jnp.float32),
)


def kernel(center, context, negatives, W_in, W_out):
    center = center.astype(jnp.int32)
    context = context.astype(jnp.int32)
    neg_flat = negatives.astype(jnp.int32).reshape(-1)
    pos, negs = _sc_scores(center, context, neg_flat, W_in, W_out)
    loss = _tc_loss(pos.reshape(128, 128), negs.reshape(B * K // 128, 128))
    return loss[0, 0]
